# baseline (speedup 1.0000x reference)
import functools

import jax
import jax.numpy as jnp
from jax import lax
from jax.experimental import pallas as pl
from jax.experimental.pallas import tpu as pltpu
from jax.experimental.pallas import tpu_sc as plsc

_NEG_INF = -1e30


def _gating_kernel(x_ref, wg_ref, bg_ref,
                   pos0_ref, pos1_ref, w0_ref, w1_ref, st_ref, se_ref,
                   cnt_scr, *, n_experts, topk, n_tiles, ffn_tile,
                   cap_tiles, n_slots):
    i = pl.program_id(0)
    x = x_ref[...]
    t = x.shape[0]
    scores = jnp.dot(x, wg_ref[...], preferred_element_type=jnp.float32)
    scores = scores + bg_ref[...][None, :]

    idx = lax.broadcasted_iota(jnp.int32, (t, n_experts), 1)
    big = jnp.int32(n_experts + 1)
    masked = scores
    sels = []
    for _ in range(topk):
        m = jnp.max(masked, axis=1, keepdims=True)
        is_max = masked == m
        first = jnp.min(jnp.where(is_max, idx, big), axis=1, keepdims=True)
        sel = idx == first
        sels.append(sel)
        masked = jnp.where(sel, _NEG_INF, masked)
    sel0, sel1 = sels
    union = jnp.logical_or(sel0, sel1)

    smax = jnp.max(scores, axis=1, keepdims=True)
    ex = jnp.exp(scores - smax)
    sm = ex / jnp.sum(ex, axis=1, keepdims=True)
    w = jnp.where(union, sm, 0.0)
    w = w / (jnp.sum(w, axis=1, keepdims=True) + 1e-8)

    @pl.when(i == 0)
    def _init():
        cnt_scr[...] = jnp.zeros_like(cnt_scr)

    base = cnt_scr[...]
    mu = union.astype(jnp.float32)
    tri = (lax.broadcasted_iota(jnp.int32, (t, t), 0) >
           lax.broadcasted_iota(jnp.int32, (t, t), 1)).astype(jnp.float32)
    rank = jnp.dot(tri, mu, preferred_element_type=jnp.float32)
    cap = jnp.float32(cap_tiles * ffn_tile)
    evec = lax.broadcasted_iota(
        jnp.int32, (1, n_experts), 1).astype(jnp.float32) * cap
    dest = evec + base + rank

    pos0 = jnp.sum(jnp.where(sel0, dest, 0.0), axis=1)
    pos1 = jnp.sum(jnp.where(sel1, dest, 0.0), axis=1)
    pos0_ref[...] = pos0.astype(jnp.int32).reshape(1, 1, t)
    pos1_ref[...] = pos1.astype(jnp.int32).reshape(1, 1, t)
    w0_ref[...] = jnp.sum(jnp.where(sel0, w, 0.0), axis=1).reshape(1, 1, t)
    w1_ref[...] = jnp.sum(jnp.where(sel1, w, 0.0), axis=1).reshape(1, 1, t)

    cnt_scr[...] = base + jnp.sum(mu, axis=0, keepdims=True)

    @pl.when(i == n_tiles - 1)
    def _slots():
        counts = cnt_scr[...].astype(jnp.int32)
        ntile = (counts + (ffn_tile - 1)) // ffn_tile
        ntf = jnp.broadcast_to(ntile.astype(jnp.float32), (n_experts, n_experts))
        r8 = lax.broadcasted_iota(jnp.int32, (n_experts, n_experts), 0)
        c8 = lax.broadcasted_iota(jnp.int32, (n_experts, n_experts), 1)
        s_incl = jnp.sum(jnp.where(c8 <= r8, ntf, 0.0), axis=1,
                         keepdims=True)
        s_excl = jnp.sum(jnp.where(c8 < r8, ntf, 0.0), axis=1, keepdims=True)
        s_iota = lax.broadcasted_iota(
            jnp.int32, (1, n_slots), 1).astype(jnp.float32)
        ge = (jnp.broadcast_to(s_iota, (n_experts, n_slots)) >=
              jnp.broadcast_to(s_incl, (n_experts, n_slots)))
        e_of = jnp.sum(ge.astype(jnp.float32), axis=0, keepdims=True)
        total = jnp.max(s_incl)
        valid = s_iota < total
        e_ofc = jnp.minimum(e_of, jnp.float32(n_experts - 1))
        er = lax.broadcasted_iota(
            jnp.int32, (n_experts, n_slots), 0).astype(jnp.float32)
        eq = (jnp.broadcast_to(e_ofc, (n_experts, n_slots)) == er)
        sx_of = jnp.sum(jnp.where(eq, jnp.broadcast_to(s_excl,
                                                       (n_experts, n_slots)),
                                  0.0), axis=0, keepdims=True)
        tile_within = s_iota - sx_of
        st = jnp.where(valid, e_ofc * cap_tiles + tile_within, 0.0)
        se = jnp.where(valid, e_ofc, 0.0)
        st_ref[...] = st.astype(jnp.int32)
        se_ref[...] = se.astype(jnp.int32)


def _run_gating(x, Wg, bg, *, ffn_tile, cap_tiles, n_slots):
    n, d = x.shape
    e = Wg.shape[1]
    t = min(512, n)
    nt = n // t
    body = functools.partial(_gating_kernel, n_experts=e, topk=2, n_tiles=nt,
                             ffn_tile=ffn_tile, cap_tiles=cap_tiles,
                             n_slots=n_slots)
    outs = pl.pallas_call(
        body,
        grid=(nt,),
        in_specs=[
            pl.BlockSpec((t, d), lambda i: (i, 0)),
            pl.BlockSpec((d, e), lambda i: (0, 0)),
            pl.BlockSpec((e,), lambda i: (0,)),
        ],
        out_specs=[
            pl.BlockSpec((1, 1, t), lambda i: (i, 0, 0)),
            pl.BlockSpec((1, 1, t), lambda i: (i, 0, 0)),
            pl.BlockSpec((1, 1, t), lambda i: (i, 0, 0)),
            pl.BlockSpec((1, 1, t), lambda i: (i, 0, 0)),
            pl.BlockSpec((1, n_slots), lambda i: (0, 0)),
            pl.BlockSpec((1, n_slots), lambda i: (0, 0)),
        ],
        out_shape=[
            jax.ShapeDtypeStruct((nt, 1, t), jnp.int32),
            jax.ShapeDtypeStruct((nt, 1, t), jnp.int32),
            jax.ShapeDtypeStruct((nt, 1, t), jnp.float32),
            jax.ShapeDtypeStruct((nt, 1, t), jnp.float32),
            jax.ShapeDtypeStruct((1, n_slots), jnp.int32),
            jax.ShapeDtypeStruct((1, n_slots), jnp.int32),
        ],
        scratch_shapes=[pltpu.VMEM((1, e), jnp.float32)],
        compiler_params=pltpu.CompilerParams(
            dimension_semantics=("arbitrary",),
        ),
    )(x, Wg, bg)
    return outs


def _make_dispatch(n, d, rows_out):
    info = plsc.get_sparse_core_info()
    nc, ns = info.num_cores, info.num_subcores
    nw = nc * ns
    chunk = n // nw
    nb = chunk // 16
    mesh = plsc.VectorSubcoreMesh(core_axis_name="c", subcore_axis_name="s")

    @functools.partial(
        pl.kernel, mesh=mesh,
        out_type=jax.ShapeDtypeStruct((rows_out, d), jnp.float32),
        scratch_types=[
            pltpu.VMEM((nb, 16), jnp.int32),
            pltpu.VMEM((nb, 16), jnp.int32),
            pltpu.VMEM((16, d), jnp.float32),
            pltpu.SemaphoreType.DMA,
        ],
    )
    def dispatch(x_hbm, p0_hbm, p1_hbm, xs_hbm, i0v, i1v, xbuf, sem):
        wid = lax.axis_index("s") * nc + lax.axis_index("c")
        base = wid * chunk
        pltpu.sync_copy(p0_hbm.at[pl.ds(wid * nb, nb), :], i0v)
        pltpu.sync_copy(p1_hbm.at[pl.ds(wid * nb, nb), :], i1v)

        def body(j, carry):
            pltpu.sync_copy(x_hbm.at[pl.ds(base + j * 16, 16), :], xbuf)
            c0 = pltpu.async_copy(xbuf, xs_hbm.at[i0v.at[j]], sem)
            c1 = pltpu.async_copy(xbuf, xs_hbm.at[i1v.at[j]], sem)
            c0.wait()
            c1.wait()
            return carry

        lax.fori_loop(0, nb, body, 0)

    return dispatch


def _ffn_kernel(st_ref, se_ref, x_ref, w1_ref, b1_ref, w2_ref, b2_ref, y_ref):
    xb = x_ref[...].astype(jnp.bfloat16)
    h = jnp.dot(xb, w1_ref[0], preferred_element_type=jnp.float32)
    h = jnp.maximum(h + b1_ref[0], 0.0)
    y = jnp.dot(h.astype(jnp.bfloat16), w2_ref[0],
                preferred_element_type=jnp.float32)
    y_ref[...] = y + b2_ref[0]


def _run_ffn(x_sorted, W1, b1, W2, b2, slot_tile, slot_expert, *, n_slots_run,
             ffn_tile):
    rows, d = x_sorted.shape
    e, _, h = W1.shape
    o = W2.shape[2]
    grid_spec = pltpu.PrefetchScalarGridSpec(
        num_scalar_prefetch=2,
        grid=(n_slots_run,),
        in_specs=[
            pl.BlockSpec((ffn_tile, d), lambda i, st, se: (st[i], 0)),
            pl.BlockSpec((1, d, h), lambda i, st, se: (se[i], 0, 0)),
            pl.BlockSpec((1, 1, h), lambda i, st, se: (se[i], 0, 0)),
            pl.BlockSpec((1, h, o), lambda i, st, se: (se[i], 0, 0)),
            pl.BlockSpec((1, 1, o), lambda i, st, se: (se[i], 0, 0)),
        ],
        out_specs=pl.BlockSpec((ffn_tile, o), lambda i, st, se: (st[i], 0)),
    )
    return pl.pallas_call(
        _ffn_kernel,
        grid_spec=grid_spec,
        out_shape=jax.ShapeDtypeStruct((rows, o), jnp.float32),
        compiler_params=pltpu.CompilerParams(
            dimension_semantics=("arbitrary",),
        ),
    )(slot_tile, slot_expert, x_sorted, W1, b1.reshape(e, 1, h), W2,
      b2.reshape(e, 1, o))


def _make_combine(n, o, rows_in):
    info = plsc.get_sparse_core_info()
    nc, ns = info.num_cores, info.num_subcores
    nw = nc * ns
    chunk = n // nw
    nb = chunk // 16
    nvec = o // 16
    mesh = plsc.VectorSubcoreMesh(core_axis_name="c", subcore_axis_name="s")

    @functools.partial(
        pl.kernel, mesh=mesh,
        out_type=jax.ShapeDtypeStruct((n, o), jnp.float32),
        scratch_types=[
            pltpu.VMEM((chunk,), jnp.int32),
            pltpu.VMEM((chunk,), jnp.int32),
            pltpu.VMEM((chunk,), jnp.float32),
            pltpu.VMEM((chunk,), jnp.float32),
            pltpu.VMEM((16, o), jnp.float32),
            pltpu.VMEM((16, o), jnp.float32),
            pltpu.VMEM((16, o), jnp.float32),
            pltpu.SemaphoreType.DMA,
        ],
        compiler_params=pltpu.CompilerParams(needs_layout_passes=False),
    )
    def combine(y_hbm, p0_hbm, p1_hbm, w0_hbm, w1_hbm, out_hbm,
                p0v, p1v, w0v, w1v, r0, r1, ob, sem):
        wid = lax.axis_index("s") * nc + lax.axis_index("c")
        base = wid * chunk
        pltpu.sync_copy(p0_hbm.at[pl.ds(base, chunk)], p0v)
        pltpu.sync_copy(p1_hbm.at[pl.ds(base, chunk)], p1v)
        pltpu.sync_copy(w0_hbm.at[pl.ds(base, chunk)], w0v)
        pltpu.sync_copy(w1_hbm.at[pl.ds(base, chunk)], w1v)

        def body(j, carry):
            g0 = pltpu.async_copy(y_hbm.at[p0v.at[pl.ds(j * 16, 16)]], r0, sem)
            g1 = pltpu.async_copy(y_hbm.at[p1v.at[pl.ds(j * 16, 16)]], r1, sem)
            g0.wait()
            g1.wait()

            def tok(ti, c2):
                iv = jnp.zeros((16,), jnp.int32) + (j * 16 + ti)
                wv0 = plsc.load_gather(w0v, [iv])
                wv1 = plsc.load_gather(w1v, [iv])
                for v in range(nvec):
                    sl = pl.ds(v * 16, 16)
                    ob[ti, sl] = wv0 * r0[ti, sl] + wv1 * r1[ti, sl]
                return c2

            lax.fori_loop(0, 16, tok, 0)
            pltpu.sync_copy(ob, out_hbm.at[pl.ds(base + j * 16, 16), :])
            return carry

        lax.fori_loop(0, nb, body, 0)

    return combine


def kernel(x, Wg, bg, W1, b1, W2, b2):
    n, d = x.shape
    e, _, h = W1.shape
    o = W2.shape[2]
    ffn_tile = min(256, n)
    cap_tiles = n // ffn_tile
    n_slots_run = (2 * n) // ffn_tile + e
    n_slots = 128
    rows = e * cap_tiles * ffn_tile

    pos0, pos1, w0, w1, slot_tile, slot_expert = _run_gating(
        x, Wg, bg, ffn_tile=ffn_tile, cap_tiles=cap_tiles, n_slots=n_slots)

    p0 = pos0.reshape(n)
    p1 = pos1.reshape(n)
    w0 = w0.reshape(n)
    w1 = w1.reshape(n)
    st = slot_tile.reshape(n_slots)
    se = slot_expert.reshape(n_slots)

    x_sorted = _make_dispatch(n, d, rows)(
        x, p0.reshape(n // 16, 16), p1.reshape(n // 16, 16))

    y_sorted = _run_ffn(x_sorted, W1.astype(jnp.bfloat16), b1,
                        W2.astype(jnp.bfloat16), b2, st, se,
                        n_slots_run=n_slots_run, ffn_tile=ffn_tile)

    out = _make_combine(n, o, rows)(y_sorted, p0, p1, w0, w1)
    return out


# speedup vs baseline: 1.1699x; 1.1699x over previous
import functools

import jax
import jax.numpy as jnp
from jax import lax
from jax.experimental import pallas as pl
from jax.experimental.pallas import tpu as pltpu
from jax.experimental.pallas import tpu_sc as plsc

_NEG_INF = -1e30


def _gating_kernel(x_ref, wg_ref, bg_ref,
                   pos0_ref, pos1_ref, w0_ref, w1_ref, st_ref, se_ref,
                   cnt_scr, *, n_experts, topk, n_tiles, ffn_tile,
                   cap_tiles, n_slots):
    i = pl.program_id(0)
    x = x_ref[...]
    t = x.shape[0]
    scores = jnp.dot(x, wg_ref[...], preferred_element_type=jnp.float32)
    scores = scores + bg_ref[...][None, :]

    idx = lax.broadcasted_iota(jnp.int32, (t, n_experts), 1)
    big = jnp.int32(n_experts + 1)
    masked = scores
    sels = []
    for _ in range(topk):
        m = jnp.max(masked, axis=1, keepdims=True)
        is_max = masked == m
        first = jnp.min(jnp.where(is_max, idx, big), axis=1, keepdims=True)
        sel = idx == first
        sels.append(sel)
        masked = jnp.where(sel, _NEG_INF, masked)
    sel0, sel1 = sels
    union = jnp.logical_or(sel0, sel1)

    smax = jnp.max(scores, axis=1, keepdims=True)
    ex = jnp.exp(scores - smax)
    sm = ex / jnp.sum(ex, axis=1, keepdims=True)
    w = jnp.where(union, sm, 0.0)
    w = w / (jnp.sum(w, axis=1, keepdims=True) + 1e-8)

    @pl.when(i == 0)
    def _init():
        cnt_scr[...] = jnp.zeros_like(cnt_scr)

    base = cnt_scr[...]
    mu = union.astype(jnp.float32)
    tri = (lax.broadcasted_iota(jnp.int32, (t, t), 0) >
           lax.broadcasted_iota(jnp.int32, (t, t), 1)).astype(jnp.float32)
    rank = jnp.dot(tri, mu, preferred_element_type=jnp.float32)
    cap = jnp.float32(cap_tiles * ffn_tile)
    evec = lax.broadcasted_iota(
        jnp.int32, (1, n_experts), 1).astype(jnp.float32) * cap
    dest = evec + base + rank

    pos0 = jnp.sum(jnp.where(sel0, dest, 0.0), axis=1)
    pos1 = jnp.sum(jnp.where(sel1, dest, 0.0), axis=1)
    pos0_ref[...] = pos0.astype(jnp.int32).reshape(1, 1, t)
    pos1_ref[...] = pos1.astype(jnp.int32).reshape(1, 1, t)
    w0_ref[...] = jnp.sum(jnp.where(sel0, w, 0.0), axis=1).reshape(1, 1, t)
    w1_ref[...] = jnp.sum(jnp.where(sel1, w, 0.0), axis=1).reshape(1, 1, t)

    cnt_scr[...] = base + jnp.sum(mu, axis=0, keepdims=True)

    @pl.when(i == n_tiles - 1)
    def _slots():
        counts = cnt_scr[...].astype(jnp.int32)
        ntile = (counts + (ffn_tile - 1)) // ffn_tile
        ntf = jnp.broadcast_to(ntile.astype(jnp.float32), (n_experts, n_experts))
        r8 = lax.broadcasted_iota(jnp.int32, (n_experts, n_experts), 0)
        c8 = lax.broadcasted_iota(jnp.int32, (n_experts, n_experts), 1)
        s_incl = jnp.sum(jnp.where(c8 <= r8, ntf, 0.0), axis=1,
                         keepdims=True)
        s_excl = jnp.sum(jnp.where(c8 < r8, ntf, 0.0), axis=1, keepdims=True)
        s_iota = lax.broadcasted_iota(
            jnp.int32, (1, n_slots), 1).astype(jnp.float32)
        ge = (jnp.broadcast_to(s_iota, (n_experts, n_slots)) >=
              jnp.broadcast_to(s_incl, (n_experts, n_slots)))
        e_of = jnp.sum(ge.astype(jnp.float32), axis=0, keepdims=True)
        total = jnp.max(s_incl)
        valid = s_iota < total
        e_ofc = jnp.minimum(e_of, jnp.float32(n_experts - 1))
        er = lax.broadcasted_iota(
            jnp.int32, (n_experts, n_slots), 0).astype(jnp.float32)
        eq = (jnp.broadcast_to(e_ofc, (n_experts, n_slots)) == er)
        sx_of = jnp.sum(jnp.where(eq, jnp.broadcast_to(s_excl,
                                                       (n_experts, n_slots)),
                                  0.0), axis=0, keepdims=True)
        tile_within = s_iota - sx_of
        st = jnp.where(valid, e_ofc * cap_tiles + tile_within, 0.0)
        se = jnp.where(valid, e_ofc, 0.0)
        st_ref[...] = st.astype(jnp.int32)
        se_ref[...] = se.astype(jnp.int32)


def _run_gating(x, Wg, bg, *, ffn_tile, cap_tiles, n_slots):
    n, d = x.shape
    e = Wg.shape[1]
    t = min(512, n)
    nt = n // t
    body = functools.partial(_gating_kernel, n_experts=e, topk=2, n_tiles=nt,
                             ffn_tile=ffn_tile, cap_tiles=cap_tiles,
                             n_slots=n_slots)
    outs = pl.pallas_call(
        body,
        grid=(nt,),
        in_specs=[
            pl.BlockSpec((t, d), lambda i: (i, 0)),
            pl.BlockSpec((d, e), lambda i: (0, 0)),
            pl.BlockSpec((e,), lambda i: (0,)),
        ],
        out_specs=[
            pl.BlockSpec((1, 1, t), lambda i: (i, 0, 0)),
            pl.BlockSpec((1, 1, t), lambda i: (i, 0, 0)),
            pl.BlockSpec((1, 1, t), lambda i: (i, 0, 0)),
            pl.BlockSpec((1, 1, t), lambda i: (i, 0, 0)),
            pl.BlockSpec((1, n_slots), lambda i: (0, 0)),
            pl.BlockSpec((1, n_slots), lambda i: (0, 0)),
        ],
        out_shape=[
            jax.ShapeDtypeStruct((nt, 1, t), jnp.int32),
            jax.ShapeDtypeStruct((nt, 1, t), jnp.int32),
            jax.ShapeDtypeStruct((nt, 1, t), jnp.float32),
            jax.ShapeDtypeStruct((nt, 1, t), jnp.float32),
            jax.ShapeDtypeStruct((1, n_slots), jnp.int32),
            jax.ShapeDtypeStruct((1, n_slots), jnp.int32),
        ],
        scratch_shapes=[pltpu.VMEM((1, e), jnp.float32)],
        compiler_params=pltpu.CompilerParams(
            dimension_semantics=("arbitrary",),
        ),
    )(x, Wg, bg)
    return outs


_DISPATCH_BIG = 32


def _make_dispatch(n, d, rows_out):
    info = plsc.get_sparse_core_info()
    nc, ns = info.num_cores, info.num_subcores
    nw = nc * ns
    chunk = n // nw
    big = _DISPATCH_BIG
    nb = chunk // big
    mesh = plsc.VectorSubcoreMesh(core_axis_name="c", subcore_axis_name="s")

    @functools.partial(
        pl.kernel, mesh=mesh,
        out_type=jax.ShapeDtypeStruct((rows_out, d), jnp.float32),
        scratch_types=[
            pltpu.VMEM((nb, big), jnp.int32),
            pltpu.VMEM((nb, big), jnp.int32),
            pltpu.VMEM((big, d), jnp.float32),
            pltpu.VMEM((big, d), jnp.float32),
            pltpu.SemaphoreType.DMA,
            pltpu.SemaphoreType.DMA,
        ],
    )
    def dispatch(x_hbm, p0_hbm, p1_hbm, xs_hbm, i0v, i1v, xa, xb, sem_r,
                 sem_w):
        wid = lax.axis_index("s") * nc + lax.axis_index("c")
        base = wid * chunk
        pltpu.sync_copy(p0_hbm.at[pl.ds(wid * nb, nb), :], i0v)
        pltpu.sync_copy(p1_hbm.at[pl.ds(wid * nb, nb), :], i1v)

        bufs = (xa, xb)
        pend = {0: [], 1: []}
        reads = {0: pltpu.async_copy(x_hbm.at[pl.ds(base, big), :], xa,
                                     sem_r)}
        for i in range(nb):
            b = i % 2
            buf = bufs[b]
            reads[i].wait()
            c0 = pltpu.async_copy(buf, xs_hbm.at[i0v.at[i]], sem_w)
            c1 = pltpu.async_copy(buf, xs_hbm.at[i1v.at[i]], sem_w)
            pend[b] = [c0, c1]
            if i + 1 < nb:
                q = (i + 1) % 2
                for c in pend[q]:
                    c.wait()
                pend[q] = []
                reads[i + 1] = pltpu.async_copy(
                    x_hbm.at[pl.ds(base + (i + 1) * big, big), :], bufs[q],
                    sem_r)
        for b in (0, 1):
            for c in pend[b]:
                c.wait()

    return dispatch


def _ffn_kernel(st_ref, se_ref, x_ref, w1_ref, b1_ref, w2_ref, b2_ref, y_ref):
    h = jnp.dot(x_ref[...], w1_ref[0], preferred_element_type=jnp.float32)
    h = jnp.maximum(h + b1_ref[0], 0.0)
    y = jnp.dot(h, w2_ref[0], preferred_element_type=jnp.float32)
    y_ref[...] = y + b2_ref[0]


def _run_ffn(x_sorted, W1, b1, W2, b2, slot_tile, slot_expert, *, n_slots_run,
             ffn_tile):
    rows, d = x_sorted.shape
    e, _, h = W1.shape
    o = W2.shape[2]
    grid_spec = pltpu.PrefetchScalarGridSpec(
        num_scalar_prefetch=2,
        grid=(n_slots_run,),
        in_specs=[
            pl.BlockSpec((ffn_tile, d), lambda i, st, se: (st[i], 0)),
            pl.BlockSpec((1, d, h), lambda i, st, se: (se[i], 0, 0)),
            pl.BlockSpec((1, 1, h), lambda i, st, se: (se[i], 0, 0)),
            pl.BlockSpec((1, h, o), lambda i, st, se: (se[i], 0, 0)),
            pl.BlockSpec((1, 1, o), lambda i, st, se: (se[i], 0, 0)),
        ],
        out_specs=pl.BlockSpec((ffn_tile, o), lambda i, st, se: (st[i], 0)),
    )
    return pl.pallas_call(
        _ffn_kernel,
        grid_spec=grid_spec,
        out_shape=jax.ShapeDtypeStruct((rows, o), jnp.float32),
        compiler_params=pltpu.CompilerParams(
            dimension_semantics=("arbitrary",),
        ),
    )(slot_tile, slot_expert, x_sorted, W1, b1.reshape(e, 1, h), W2,
      b2.reshape(e, 1, o))


def _make_combine(n, o, rows_in):
    info = plsc.get_sparse_core_info()
    nc, ns = info.num_cores, info.num_subcores
    nw = nc * ns
    chunk = n // nw
    nb = chunk // 16
    nvec = o // 16
    mesh = plsc.VectorSubcoreMesh(core_axis_name="c", subcore_axis_name="s")

    @functools.partial(
        pl.kernel, mesh=mesh,
        out_type=jax.ShapeDtypeStruct((n, o), jnp.float32),
        scratch_types=[
            pltpu.VMEM((chunk,), jnp.int32),
            pltpu.VMEM((chunk,), jnp.int32),
            pltpu.VMEM((chunk,), jnp.float32),
            pltpu.VMEM((chunk,), jnp.float32),
            pltpu.VMEM((16, o), jnp.float32),
            pltpu.VMEM((16, o), jnp.float32),
            pltpu.VMEM((16, o), jnp.float32),
            pltpu.VMEM((16, o), jnp.float32),
            pltpu.VMEM((16, o), jnp.float32),
            pltpu.VMEM((16, o), jnp.float32),
            pltpu.SemaphoreType.DMA,
        ],
        compiler_params=pltpu.CompilerParams(needs_layout_passes=False),
    )
    def combine(y_hbm, p0_hbm, p1_hbm, w0_hbm, w1_hbm, out_hbm,
                p0v, p1v, w0v, w1v, r0a, r1a, r0b, r1b, oba, obb, sem):
        wid = lax.axis_index("s") * nc + lax.axis_index("c")
        base = wid * chunk
        pltpu.sync_copy(p0_hbm.at[pl.ds(base, chunk)], p0v)
        pltpu.sync_copy(p1_hbm.at[pl.ds(base, chunk)], p1v)
        pltpu.sync_copy(w0_hbm.at[pl.ds(base, chunk)], w0v)
        pltpu.sync_copy(w1_hbm.at[pl.ds(base, chunk)], w1v)

        def issue(j, d0, d1):
            off = j * 16
            g0 = pltpu.async_copy(y_hbm.at[p0v.at[pl.ds(off, 16)]], d0, sem)
            g1 = pltpu.async_copy(y_hbm.at[p1v.at[pl.ds(off, 16)]], d1, sem)
            return g0, g1

        def drain(d0, d1):
            pltpu.make_async_copy(y_hbm.at[p0v.at[pl.ds(0, 16)]], d0,
                                  sem).wait()
            pltpu.make_async_copy(y_hbm.at[p1v.at[pl.ds(0, 16)]], d1,
                                  sem).wait()

        def compute(j, s0, s1, ob):
            def tok(ti, c2):
                iv = jnp.zeros((16,), jnp.int32) + (j * 16 + ti)
                wv0 = plsc.load_gather(w0v, [iv])
                wv1 = plsc.load_gather(w1v, [iv])
                for v in range(nvec):
                    sl = pl.ds(v * 16, 16)
                    ob[ti, sl] = wv0 * s0[ti, sl] + wv1 * s1[ti, sl]
                return c2

            lax.fori_loop(0, 16, tok, 0)
            pltpu.sync_copy(ob, out_hbm.at[pl.ds(base + j * 16, 16), :])

        issue(0, r0a, r1a)

        def pair(i, carry):
            j0 = 2 * i
            j1 = 2 * i + 1
            issue(j1, r0b, r1b)
            drain(r0a, r1a)
            compute(j0, r0a, r1a, oba)
            jn = jnp.minimum(j0 + 2, nb - 1)
            issue(jn, r0a, r1a)
            drain(r0b, r1b)
            compute(j1, r0b, r1b, obb)
            return carry

        lax.fori_loop(0, nb // 2, pair, 0)
        drain(r0a, r1a)

    return combine


def kernel(x, Wg, bg, W1, b1, W2, b2):
    n, d = x.shape
    e, _, h = W1.shape
    o = W2.shape[2]
    ffn_tile = min(256, n)
    cap_tiles = n // ffn_tile
    n_slots_run = (2 * n) // ffn_tile + e
    n_slots = 128
    rows = e * cap_tiles * ffn_tile

    pos0, pos1, w0, w1, slot_tile, slot_expert = _run_gating(
        x, Wg, bg, ffn_tile=ffn_tile, cap_tiles=cap_tiles, n_slots=n_slots)

    p0 = pos0.reshape(n)
    p1 = pos1.reshape(n)
    w0 = w0.reshape(n)
    w1 = w1.reshape(n)
    st = slot_tile.reshape(n_slots)
    se = slot_expert.reshape(n_slots)

    x_sorted = _make_dispatch(n, d, rows)(
        x, p0.reshape(n // _DISPATCH_BIG, _DISPATCH_BIG),
        p1.reshape(n // _DISPATCH_BIG, _DISPATCH_BIG))

    y_sorted = _run_ffn(x_sorted, W1, b1, W2, b2, st, se,
                        n_slots_run=n_slots_run, ffn_tile=ffn_tile)

    out = _make_combine(n, o, rows)(y_sorted, p0, p1, w0, w1)
    return out


# speedup vs baseline: 1.1963x; 1.0226x over previous
import functools

import jax
import jax.numpy as jnp
from jax import lax
from jax.experimental import pallas as pl
from jax.experimental.pallas import tpu as pltpu
from jax.experimental.pallas import tpu_sc as plsc

_NEG_INF = -1e30


def _gating_kernel(x_ref, wg_ref, bg_ref,
                   pos0_ref, pos1_ref, w0_ref, w1_ref, st_ref, se_ref,
                   cnt_scr, *, n_experts, topk, n_tiles, ffn_tile,
                   cap_tiles, n_slots):
    i = pl.program_id(0)
    x = x_ref[...]
    t = x.shape[0]
    scores = jnp.dot(x, wg_ref[...], preferred_element_type=jnp.float32)
    scores = scores + bg_ref[...][None, :]

    idx = lax.broadcasted_iota(jnp.int32, (t, n_experts), 1)
    big = jnp.int32(n_experts + 1)
    masked = scores
    sels = []
    for _ in range(topk):
        m = jnp.max(masked, axis=1, keepdims=True)
        is_max = masked == m
        first = jnp.min(jnp.where(is_max, idx, big), axis=1, keepdims=True)
        sel = idx == first
        sels.append(sel)
        masked = jnp.where(sel, _NEG_INF, masked)
    sel0, sel1 = sels
    union = jnp.logical_or(sel0, sel1)

    smax = jnp.max(scores, axis=1, keepdims=True)
    ex = jnp.exp(scores - smax)
    sm = ex / jnp.sum(ex, axis=1, keepdims=True)
    w = jnp.where(union, sm, 0.0)
    w = w / (jnp.sum(w, axis=1, keepdims=True) + 1e-8)

    @pl.when(i == 0)
    def _init():
        cnt_scr[...] = jnp.zeros_like(cnt_scr)

    base = cnt_scr[...]
    mu = union.astype(jnp.float32)
    tri = (lax.broadcasted_iota(jnp.int32, (t, t), 0) >
           lax.broadcasted_iota(jnp.int32, (t, t), 1)).astype(jnp.float32)
    rank = jnp.dot(tri, mu, preferred_element_type=jnp.float32)
    cap = jnp.float32(cap_tiles * ffn_tile)
    evec = lax.broadcasted_iota(
        jnp.int32, (1, n_experts), 1).astype(jnp.float32) * cap
    dest = evec + base + rank

    pos0 = jnp.sum(jnp.where(sel0, dest, 0.0), axis=1)
    pos1 = jnp.sum(jnp.where(sel1, dest, 0.0), axis=1)
    pos0_ref[...] = pos0.astype(jnp.int32).reshape(1, 1, t)
    pos1_ref[...] = pos1.astype(jnp.int32).reshape(1, 1, t)
    w0_ref[...] = jnp.sum(jnp.where(sel0, w, 0.0), axis=1).reshape(1, 1, t)
    w1_ref[...] = jnp.sum(jnp.where(sel1, w, 0.0), axis=1).reshape(1, 1, t)

    cnt_scr[...] = base + jnp.sum(mu, axis=0, keepdims=True)

    @pl.when(i == n_tiles - 1)
    def _slots():
        counts = cnt_scr[...].astype(jnp.int32)
        ntile = (counts + (ffn_tile - 1)) // ffn_tile
        ntf = jnp.broadcast_to(ntile.astype(jnp.float32), (n_experts, n_experts))
        r8 = lax.broadcasted_iota(jnp.int32, (n_experts, n_experts), 0)
        c8 = lax.broadcasted_iota(jnp.int32, (n_experts, n_experts), 1)
        s_incl = jnp.sum(jnp.where(c8 <= r8, ntf, 0.0), axis=1,
                         keepdims=True)
        s_excl = jnp.sum(jnp.where(c8 < r8, ntf, 0.0), axis=1, keepdims=True)
        s_iota = lax.broadcasted_iota(
            jnp.int32, (1, n_slots), 1).astype(jnp.float32)
        ge = (jnp.broadcast_to(s_iota, (n_experts, n_slots)) >=
              jnp.broadcast_to(s_incl, (n_experts, n_slots)))
        e_of = jnp.sum(ge.astype(jnp.float32), axis=0, keepdims=True)
        total = jnp.max(s_incl)
        valid = s_iota < total
        e_ofc = jnp.minimum(e_of, jnp.float32(n_experts - 1))
        er = lax.broadcasted_iota(
            jnp.int32, (n_experts, n_slots), 0).astype(jnp.float32)
        eq = (jnp.broadcast_to(e_ofc, (n_experts, n_slots)) == er)
        sx_of = jnp.sum(jnp.where(eq, jnp.broadcast_to(s_excl,
                                                       (n_experts, n_slots)),
                                  0.0), axis=0, keepdims=True)
        tile_within = s_iota - sx_of
        st = jnp.where(valid, e_ofc * cap_tiles + tile_within, 0.0)
        se = jnp.where(valid, e_ofc, 0.0)
        st_ref[...] = st.astype(jnp.int32)
        se_ref[...] = se.astype(jnp.int32)


def _run_gating(x, Wg, bg, *, ffn_tile, cap_tiles, n_slots):
    n, d = x.shape
    e = Wg.shape[1]
    t = min(512, n)
    nt = n // t
    body = functools.partial(_gating_kernel, n_experts=e, topk=2, n_tiles=nt,
                             ffn_tile=ffn_tile, cap_tiles=cap_tiles,
                             n_slots=n_slots)
    outs = pl.pallas_call(
        body,
        grid=(nt,),
        in_specs=[
            pl.BlockSpec((t, d), lambda i: (i, 0)),
            pl.BlockSpec((d, e), lambda i: (0, 0)),
            pl.BlockSpec((e,), lambda i: (0,)),
        ],
        out_specs=[
            pl.BlockSpec((1, 1, t), lambda i: (i, 0, 0)),
            pl.BlockSpec((1, 1, t), lambda i: (i, 0, 0)),
            pl.BlockSpec((1, 1, t), lambda i: (i, 0, 0)),
            pl.BlockSpec((1, 1, t), lambda i: (i, 0, 0)),
            pl.BlockSpec((1, n_slots), lambda i: (0, 0)),
            pl.BlockSpec((1, n_slots), lambda i: (0, 0)),
        ],
        out_shape=[
            jax.ShapeDtypeStruct((nt, 1, t), jnp.int32),
            jax.ShapeDtypeStruct((nt, 1, t), jnp.int32),
            jax.ShapeDtypeStruct((nt, 1, t), jnp.float32),
            jax.ShapeDtypeStruct((nt, 1, t), jnp.float32),
            jax.ShapeDtypeStruct((1, n_slots), jnp.int32),
            jax.ShapeDtypeStruct((1, n_slots), jnp.int32),
        ],
        scratch_shapes=[pltpu.VMEM((1, e), jnp.float32)],
        compiler_params=pltpu.CompilerParams(
            dimension_semantics=("arbitrary",),
        ),
    )(x, Wg, bg)
    return outs


_DISPATCH_BIG = 32


def _make_dispatch(n, d, rows_out):
    info = plsc.get_sparse_core_info()
    nc, ns = info.num_cores, info.num_subcores
    nw = nc * ns
    chunk = n // nw
    big = _DISPATCH_BIG
    nb = chunk // big
    mesh = plsc.VectorSubcoreMesh(core_axis_name="c", subcore_axis_name="s")

    @functools.partial(
        pl.kernel, mesh=mesh,
        out_type=jax.ShapeDtypeStruct((rows_out, d), jnp.float32),
        scratch_types=[
            pltpu.VMEM((nb, big), jnp.int32),
            pltpu.VMEM((nb, big), jnp.int32),
            pltpu.VMEM((big, d), jnp.float32),
            pltpu.VMEM((big, d), jnp.float32),
            pltpu.SemaphoreType.DMA,
            pltpu.SemaphoreType.DMA,
        ],
    )
    def dispatch(x_hbm, p0_hbm, p1_hbm, xs_hbm, i0v, i1v, xa, xb, sem_r,
                 sem_w):
        wid = lax.axis_index("s") * nc + lax.axis_index("c")
        base = wid * chunk
        pltpu.sync_copy(p0_hbm.at[pl.ds(wid * nb, nb), :], i0v)
        pltpu.sync_copy(p1_hbm.at[pl.ds(wid * nb, nb), :], i1v)

        bufs = (xa, xb)
        pend = {0: [], 1: []}
        reads = {0: pltpu.async_copy(x_hbm.at[pl.ds(base, big), :], xa,
                                     sem_r)}
        for i in range(nb):
            b = i % 2
            buf = bufs[b]
            reads[i].wait()
            c0 = pltpu.async_copy(buf, xs_hbm.at[i0v.at[i]], sem_w)
            c1 = pltpu.async_copy(buf, xs_hbm.at[i1v.at[i]], sem_w)
            pend[b] = [c0, c1]
            if i + 1 < nb:
                q = (i + 1) % 2
                for c in pend[q]:
                    c.wait()
                pend[q] = []
                reads[i + 1] = pltpu.async_copy(
                    x_hbm.at[pl.ds(base + (i + 1) * big, big), :], bufs[q],
                    sem_r)
        for b in (0, 1):
            for c in pend[b]:
                c.wait()

    return dispatch


def _ffn_kernel(st_ref, se_ref, x_ref, w1_ref, b1_ref, w2_ref, b2_ref, y_ref):
    h = jnp.dot(x_ref[...], w1_ref[0], preferred_element_type=jnp.float32)
    h = jnp.maximum(h + b1_ref[0], 0.0)
    y = jnp.dot(h, w2_ref[0], preferred_element_type=jnp.float32)
    y_ref[...] = y + b2_ref[0]


def _run_ffn(x_sorted, W1, b1, W2, b2, slot_tile, slot_expert, *, n_slots_run,
             ffn_tile):
    rows, d = x_sorted.shape
    e, _, h = W1.shape
    o = W2.shape[2]
    grid_spec = pltpu.PrefetchScalarGridSpec(
        num_scalar_prefetch=2,
        grid=(n_slots_run,),
        in_specs=[
            pl.BlockSpec((ffn_tile, d), lambda i, st, se: (st[i], 0)),
            pl.BlockSpec((1, d, h), lambda i, st, se: (se[i], 0, 0)),
            pl.BlockSpec((1, 1, h), lambda i, st, se: (se[i], 0, 0)),
            pl.BlockSpec((1, h, o), lambda i, st, se: (se[i], 0, 0)),
            pl.BlockSpec((1, 1, o), lambda i, st, se: (se[i], 0, 0)),
        ],
        out_specs=pl.BlockSpec((ffn_tile, o), lambda i, st, se: (st[i], 0)),
    )
    return pl.pallas_call(
        _ffn_kernel,
        grid_spec=grid_spec,
        out_shape=jax.ShapeDtypeStruct((rows, o), jnp.float32),
        compiler_params=pltpu.CompilerParams(
            dimension_semantics=("arbitrary",),
        ),
    )(slot_tile, slot_expert, x_sorted, W1, b1.reshape(e, 1, h), W2,
      b2.reshape(e, 1, o))


def _make_combine(n, o, rows_in):
    info = plsc.get_sparse_core_info()
    nc, ns = info.num_cores, info.num_subcores
    nw = nc * ns
    chunk = n // nw
    nb = chunk // 16
    nvec = o // 16
    mesh = plsc.VectorSubcoreMesh(core_axis_name="c", subcore_axis_name="s")

    @functools.partial(
        pl.kernel, mesh=mesh,
        out_type=jax.ShapeDtypeStruct((n, o), jnp.float32),
        scratch_types=[
            pltpu.VMEM((chunk,), jnp.int32),
            pltpu.VMEM((chunk,), jnp.int32),
            pltpu.VMEM((chunk,), jnp.float32),
            pltpu.VMEM((chunk,), jnp.float32),
            pltpu.VMEM((16, o), jnp.float32),
            pltpu.VMEM((16, o), jnp.float32),
            pltpu.VMEM((16, o), jnp.float32),
            pltpu.VMEM((16, o), jnp.float32),
            pltpu.VMEM((16, o), jnp.float32),
            pltpu.VMEM((16, o), jnp.float32),
            pltpu.SemaphoreType.DMA,
        ],
        compiler_params=pltpu.CompilerParams(needs_layout_passes=False),
    )
    def combine(y_hbm, p0_hbm, p1_hbm, w0_hbm, w1_hbm, out_hbm,
                p0v, p1v, w0v, w1v, r0a, r1a, r0b, r1b, oba, obb, sem):
        wid = lax.axis_index("s") * nc + lax.axis_index("c")
        base = wid * chunk
        pltpu.sync_copy(p0_hbm.at[pl.ds(base, chunk)], p0v)
        pltpu.sync_copy(p1_hbm.at[pl.ds(base, chunk)], p1v)
        pltpu.sync_copy(w0_hbm.at[pl.ds(base, chunk)], w0v)
        pltpu.sync_copy(w1_hbm.at[pl.ds(base, chunk)], w1v)

        def issue(j, d0, d1):
            off = j * 16
            g0 = pltpu.async_copy(y_hbm.at[p0v.at[pl.ds(off, 16)]], d0, sem)
            g1 = pltpu.async_copy(y_hbm.at[p1v.at[pl.ds(off, 16)]], d1, sem)
            return g0, g1

        def drain(d0, d1):
            pltpu.make_async_copy(y_hbm.at[p0v.at[pl.ds(0, 16)]], d0,
                                  sem).wait()
            pltpu.make_async_copy(y_hbm.at[p1v.at[pl.ds(0, 16)]], d1,
                                  sem).wait()

        def compute(j, s0, s1, ob):
            def tok(ti, c2):
                iv = jnp.zeros((16,), jnp.int32) + (j * 16 + ti)
                wv0 = plsc.load_gather(w0v, [iv])
                wv1 = plsc.load_gather(w1v, [iv])
                for v in range(nvec):
                    sl = pl.ds(v * 16, 16)
                    ob[ti, sl] = wv0 * s0[ti, sl] + wv1 * s1[ti, sl]
                return c2

            lax.fori_loop(0, 16, tok, 0)
            pltpu.sync_copy(ob, out_hbm.at[pl.ds(base + j * 16, 16), :])

        issue(0, r0a, r1a)

        def pair(i, carry):
            j0 = 2 * i
            j1 = 2 * i + 1
            issue(j1, r0b, r1b)
            drain(r0a, r1a)
            compute(j0, r0a, r1a, oba)
            jn = jnp.minimum(j0 + 2, nb - 1)
            issue(jn, r0a, r1a)
            drain(r0b, r1b)
            compute(j1, r0b, r1b, obb)
            return carry

        lax.fori_loop(0, nb // 2, pair, 0)
        drain(r0a, r1a)

    return combine


def kernel(x, Wg, bg, W1, b1, W2, b2):
    n, d = x.shape
    e, _, h = W1.shape
    o = W2.shape[2]
    ffn_tile = min(512, n)
    cap_tiles = n // ffn_tile
    n_slots_run = (2 * n) // ffn_tile + e
    n_slots = 128
    rows = e * cap_tiles * ffn_tile

    pos0, pos1, w0, w1, slot_tile, slot_expert = _run_gating(
        x, Wg, bg, ffn_tile=ffn_tile, cap_tiles=cap_tiles, n_slots=n_slots)

    p0 = pos0.reshape(n)
    p1 = pos1.reshape(n)
    w0 = w0.reshape(n)
    w1 = w1.reshape(n)
    st = slot_tile.reshape(n_slots)
    se = slot_expert.reshape(n_slots)

    x_sorted = _make_dispatch(n, d, rows)(
        x, p0.reshape(n // _DISPATCH_BIG, _DISPATCH_BIG),
        p1.reshape(n // _DISPATCH_BIG, _DISPATCH_BIG))

    y_sorted = _run_ffn(x_sorted, W1, b1, W2, b2, st, se,
                        n_slots_run=n_slots_run, ffn_tile=ffn_tile)

    out = _make_combine(n, o, rows)(y_sorted, p0, p1, w0, w1)
    return out
